# baseline (speedup 1.0000x reference)
import functools

import jax
import jax.numpy as jnp
from jax import lax
from jax.experimental import pallas as pl
from jax.experimental.pallas import tpu as pltpu
from jax.experimental.pallas import tpu_sc as plsc

V = 100000
D = 128
B = 16384
CHUNK = 128


@functools.lru_cache(maxsize=None)
def _build_gather():
    info = plsc.get_sparse_core_info()
    nc, ns = info.num_cores, info.num_subcores
    nw = nc * ns
    rows_per_w = B // nw
    chunks = rows_per_w // CHUNK

    mesh = plsc.VectorSubcoreMesh(core_axis_name="c", subcore_axis_name="s")

    @functools.partial(
        pl.kernel,
        mesh=mesh,
        out_type=(
            jax.ShapeDtypeStruct((B, D), jnp.float32),
            jax.ShapeDtypeStruct((B, D), jnp.float32),
        ),
        scratch_types=[
            pltpu.VMEM((rows_per_w,), jnp.int32),
            pltpu.VMEM((rows_per_w,), jnp.int32),
        ] + [pltpu.VMEM((CHUNK, D), jnp.float32) for _ in range(6)]
          + [pltpu.SemaphoreType.DMA for _ in range(6)]
          + [pltpu.SemaphoreType.DMA for _ in range(6)],
    )
    def gather(semb, temb, xi_hbm, yi_hbm, out_x, out_y,
               xi_v, yi_v, r0, r1, r2, r3, r4, r5,
               g0, g1, g2, g3, g4, g5, t0, t1, t2, t3, t4, t5):
        nbuf = 6
        bufs = (r0, r1, r2, r3, r4, r5)
        gsems = (g0, g1, g2, g3, g4, g5)
        ssems = (t0, t1, t2, t3, t4, t5)
        wid = lax.axis_index("s") * nc + lax.axis_index("c")
        base = wid * rows_per_w
        ix = pltpu.async_copy(xi_hbm.at[pl.ds(base, rows_per_w)], xi_v, g0)
        iy = pltpu.async_copy(yi_hbm.at[pl.ds(base, rows_per_w)], yi_v, g1)
        ix.wait()
        iy.wait()
        tasks = ([(semb, xi_v, out_x, j) for j in range(chunks)]
                 + [(temb, yi_v, out_y, j) for j in range(chunks)])
        n = len(tasks)
        def start(i):
            tbl, iv, _, j = tasks[i]
            return pltpu.async_copy(
                tbl.at[iv.at[pl.ds(j * CHUNK, CHUNK)]],
                bufs[i % nbuf], gsems[i % nbuf])
        copies = {i: start(i) for i in range(min(nbuf, n))}
        stores = {}
        for i, (tbl, iv, out, j) in enumerate(tasks):
            copies[i].wait()
            stores[i] = pltpu.async_copy(
                bufs[i % nbuf], out.at[pl.ds(base + j * CHUNK, CHUNK)],
                ssems[i % nbuf])
            lag = i - (nbuf - 2)
            if 0 <= lag and lag + nbuf < n:
                stores[lag].wait()
                copies[lag + nbuf] = start(lag + nbuf)
        for i in sorted(stores):
            if i + nbuf >= n:
                stores[i].wait()

    return gather


def _project(xg, yg, m):
    blk = 8192

    def body(m_ref, x_ref, y_ref, ox_ref, oy_ref):
        mm = m_ref[...]
        dn = (((1,), (1,)), ((), ()))
        ox_ref[...] = lax.dot_general(x_ref[...], mm, dn,
                                      preferred_element_type=jnp.float32)
        oy_ref[...] = lax.dot_general(y_ref[...], mm, dn,
                                      preferred_element_type=jnp.float32)

    return pl.pallas_call(
        body,
        grid=(B // blk,),
        in_specs=[
            pl.BlockSpec((D, D), lambda i: (0, 0)),
            pl.BlockSpec((blk, D), lambda i: (i, 0)),
            pl.BlockSpec((blk, D), lambda i: (i, 0)),
        ],
        out_specs=[
            pl.BlockSpec((blk, D), lambda i: (i, 0)),
            pl.BlockSpec((blk, D), lambda i: (i, 0)),
        ],
        out_shape=[jax.ShapeDtypeStruct((B, D), jnp.float32)] * 2,
    )(m, xg, yg)


def kernel(x_idx, y_idx, semb, temb, M):
    xi = x_idx.astype(jnp.int32)
    yi = y_idx.astype(jnp.int32)
    xg, yg = _build_gather()(semb, temb, xi, yi)
    return tuple(_project(xg, yg, M))


# speedup vs baseline: 1.0269x; 1.0269x over previous
import functools

import jax
import jax.numpy as jnp
from jax import lax
from jax.experimental import pallas as pl
from jax.experimental.pallas import tpu as pltpu
from jax.experimental.pallas import tpu_sc as plsc

V = 100000
D = 128
B = 16384
CHUNK = 128


@functools.lru_cache(maxsize=None)
def _build_gather():
    info = plsc.get_sparse_core_info()
    nc, ns = info.num_cores, info.num_subcores
    nw = nc * ns
    rows_per_w = B // nw
    chunks = rows_per_w // CHUNK

    mesh = plsc.VectorSubcoreMesh(core_axis_name="c", subcore_axis_name="s")

    @functools.partial(
        pl.kernel,
        mesh=mesh,
        out_type=(
            jax.ShapeDtypeStruct((B, D), jnp.float32),
            jax.ShapeDtypeStruct((B, D), jnp.float32),
        ),
        scratch_types=[
            pltpu.VMEM((rows_per_w,), jnp.int32),
            pltpu.VMEM((rows_per_w,), jnp.int32),
        ] + [pltpu.VMEM((CHUNK, D), jnp.float32) for _ in range(4)]
          + [pltpu.SemaphoreType.DMA for _ in range(4)],
    )
    def gather(semb, temb, xi_hbm, yi_hbm, out_x, out_y,
               xi_v, yi_v, r0, r1, r2, r3, s0, s1, s2, s3):
        nbuf = 4
        bufs, sems = (r0, r1, r2, r3), (s0, s1, s2, s3)
        wid = lax.axis_index("s") * nc + lax.axis_index("c")
        base = wid * rows_per_w
        ix = pltpu.async_copy(xi_hbm.at[pl.ds(base, rows_per_w)], xi_v, s0)
        iy = pltpu.async_copy(yi_hbm.at[pl.ds(base, rows_per_w)], yi_v, s1)
        ix.wait()
        iy.wait()
        tasks = ([(semb, xi_v, out_x, j) for j in range(chunks)]
                 + [(temb, yi_v, out_y, j) for j in range(chunks)])
        def start(i):
            tbl, iv, _, j = tasks[i]
            return pltpu.async_copy(
                tbl.at[iv.at[pl.ds(j * CHUNK, CHUNK)]],
                bufs[i % nbuf], sems[i % nbuf])
        copies = {i: start(i) for i in range(min(nbuf, len(tasks)))}
        for i, (tbl, iv, out, j) in enumerate(tasks):
            copies[i].wait()
            pltpu.sync_copy(bufs[i % nbuf],
                            out.at[pl.ds(base + j * CHUNK, CHUNK)])
            if i + nbuf < len(tasks):
                copies[i + nbuf] = start(i + nbuf)

    return gather


def _project(xg, yg, m):
    blk = 8192

    def body(m_ref, x_ref, y_ref, ox_ref, oy_ref):
        mm = m_ref[...]
        dn = (((1,), (1,)), ((), ()))
        ox_ref[...] = lax.dot_general(x_ref[...], mm, dn,
                                      preferred_element_type=jnp.float32)
        oy_ref[...] = lax.dot_general(y_ref[...], mm, dn,
                                      preferred_element_type=jnp.float32)

    return pl.pallas_call(
        body,
        grid=(B // blk,),
        in_specs=[
            pl.BlockSpec((D, D), lambda i: (0, 0)),
            pl.BlockSpec((blk, D), lambda i: (i, 0)),
            pl.BlockSpec((blk, D), lambda i: (i, 0)),
        ],
        out_specs=[
            pl.BlockSpec((blk, D), lambda i: (i, 0)),
            pl.BlockSpec((blk, D), lambda i: (i, 0)),
        ],
        out_shape=[jax.ShapeDtypeStruct((B, D), jnp.float32)] * 2,
    )(m, xg, yg)


def kernel(x_idx, y_idx, semb, temb, M):
    xi = x_idx.astype(jnp.int32)
    yi = y_idx.astype(jnp.int32)
    xg, yg = _build_gather()(semb, temb, xi, yi)
    return tuple(_project(xg, yg, M))
